# Initial kernel scaffold; baseline (speedup 1.0000x reference)
import jax
import jax.numpy as jnp
from jax.experimental import pallas as pl


def kernel(adjacency, feature, W1, b1, W2, b2):
    raise NotImplementedError("write your pallas kernel here")


# speedup vs baseline: 8.8778x; 8.8778x over previous
import functools

import jax
import jax.numpy as jnp
from jax import lax
from jax.experimental import pallas as pl
from jax.experimental.pallas import tpu as pltpu
from jax.experimental.pallas import tpu_sc as plsc

N = 10000
E = 320000
D_IN = 128
D_HID = 16
D_OUT = 7

NC = 2
NS = 16
NW = NC * NS
E_PER_W = E // NW
CHUNK = 128
NFULL = E_PER_W // CHUNK
TAIL = E_PER_W - NFULL * CHUNK
ROWS_PER_TILE = 640
LAST_ROW0 = N - ROWS_PER_TILE


_mesh = plsc.VectorSubcoreMesh(core_axis_name="c", subcore_axis_name="s")


@functools.partial(
    pl.kernel,
    mesh=_mesh,
    out_type=jax.ShapeDtypeStruct((NC, N, D_HID), jnp.float32),
    compiler_params=pltpu.CompilerParams(use_tc_tiling_on_sc=False),
    scratch_types=[
        pltpu.VMEM((CHUNK,), jnp.int32),
        pltpu.VMEM((CHUNK,), jnp.int32),
        pltpu.VMEM((CHUNK, D_HID), jnp.float32),
        pltpu.VMEM((TAIL,), jnp.int32),
        pltpu.VMEM((TAIL,), jnp.int32),
        pltpu.VMEM((TAIL, D_HID), jnp.float32),
        pltpu.VMEM((ROWS_PER_TILE, D_HID), jnp.float32),
        pltpu.VMEM_SHARED((N, D_HID), jnp.float32),
        pltpu.SemaphoreType.DMA,
    ],
)
def _spmm_sc(src_hbm, dst_hbm, table_hbm, out_hbm,
             src_v, dst_v, rows_v, src_t, dst_t, rows_t,
             stage_v, acc_sh, sem):
    c = lax.axis_index("c")
    s = lax.axis_index("s")
    w = c * NS + s

    def _zero(i, carry):
        stage_v[i, :] = jnp.zeros((D_HID,), jnp.float32)
        return carry

    lax.fori_loop(0, ROWS_PER_TILE, _zero, 0)
    row0 = pl.multiple_of(lax.min(s * ROWS_PER_TILE, LAST_ROW0), 8)
    pltpu.sync_copy(stage_v, acc_sh.at[pl.ds(row0, ROWS_PER_TILE)])
    plsc.subcore_barrier()

    base = w * E_PER_W

    def _chunk(i, carry):
        off = base + i * CHUNK
        pltpu.sync_copy(src_hbm.at[pl.ds(off, CHUNK)], src_v)
        pltpu.sync_copy(dst_hbm.at[pl.ds(off, CHUNK)], dst_v)
        pltpu.async_copy(table_hbm.at[src_v], rows_v, sem).wait()
        pltpu.sync_copy(rows_v, acc_sh.at[dst_v], add=True)
        return carry

    lax.fori_loop(0, NFULL, _chunk, 0)

    off = base + NFULL * CHUNK
    pltpu.sync_copy(src_hbm.at[pl.ds(off, TAIL)], src_t)
    pltpu.sync_copy(dst_hbm.at[pl.ds(off, TAIL)], dst_t)
    pltpu.async_copy(table_hbm.at[src_t], rows_t, sem).wait()
    pltpu.sync_copy(rows_t, acc_sh.at[dst_t], add=True)

    plsc.subcore_barrier()
    pltpu.sync_copy(acc_sh.at[pl.ds(row0, ROWS_PER_TILE)],
                    out_hbm.at[c, pl.ds(row0, ROWS_PER_TILE)])


def _mm1_body(x_ref, w_ref, o_ref):
    o_ref[...] = jnp.dot(x_ref[...], w_ref[...],
                         preferred_element_type=jnp.float32)


def _tc_mm1(x, w):
    return pl.pallas_call(
        _mm1_body,
        out_shape=jax.ShapeDtypeStruct((N, D_HID), jnp.float32),
    )(x, w)


def _mid_body(p_ref, b1_ref, w2_ref, o_ref):
    h = jnp.maximum(p_ref[0] + p_ref[1] + b1_ref[...][None, :], 0.0)
    o_ref[...] = jnp.dot(h, w2_ref[...], preferred_element_type=jnp.float32)


def _tc_mid(parts, b1, w2p):
    return pl.pallas_call(
        _mid_body,
        out_shape=jax.ShapeDtypeStruct((N, D_HID), jnp.float32),
    )(parts, b1, w2p)


def _fin_body(p_ref, b2_ref, o_ref):
    o_ref[...] = p_ref[0] + p_ref[1] + b2_ref[...][None, :]


def _tc_fin(parts, b2p):
    return pl.pallas_call(
        _fin_body,
        out_shape=jax.ShapeDtypeStruct((N, D_HID), jnp.float32),
    )(parts, b2p)


def kernel(adjacency, feature, W1, b1, W2, b2):
    adj = adjacency.astype(jnp.int32)
    src = adj[0]
    dst = adj[1]

    support1 = _tc_mm1(feature, W1)
    parts1 = _spmm_sc(src, dst, support1)

    w2p = jnp.pad(W2, ((0, 0), (0, D_HID - D_OUT)))
    support2 = _tc_mid(parts1, b1, w2p)
    parts2 = _spmm_sc(src, dst, support2)

    b2p = jnp.pad(b2, (0, D_HID - D_OUT))
    out16 = _tc_fin(parts2, b2p)
    return out16[:, :D_OUT]


# speedup vs baseline: 19.5018x; 2.1967x over previous
import functools

import jax
import jax.numpy as jnp
from jax import lax
from jax.experimental import pallas as pl
from jax.experimental.pallas import tpu as pltpu
from jax.experimental.pallas import tpu_sc as plsc

N = 10000
E = 320000
D_IN = 128
D_HID = 16
D_OUT = 7

NC = 2
NS = 16
NW = NC * NS
E_PER_W = E // NW
CHUNK = 80
K = 5
NGROUPS = E_PER_W // (CHUNK * K)
ROWS_PER_TILE = 640
LAST_ROW0 = N - ROWS_PER_TILE


_mesh = plsc.VectorSubcoreMesh(core_axis_name="c", subcore_axis_name="s")


@functools.partial(
    pl.kernel,
    mesh=_mesh,
    out_type=jax.ShapeDtypeStruct((NC, N, D_HID), jnp.float32),
    compiler_params=pltpu.CompilerParams(use_tc_tiling_on_sc=False),
    scratch_types=[
        pltpu.VMEM((K, CHUNK), jnp.int32),
        pltpu.VMEM((K, CHUNK), jnp.int32),
        pltpu.VMEM((K, CHUNK, D_HID), jnp.float32),
        pltpu.VMEM((ROWS_PER_TILE, D_HID), jnp.float32),
        pltpu.VMEM_SHARED((N, D_HID), jnp.float32),
        pltpu.SemaphoreType.DMA,
        pltpu.SemaphoreType.DMA,
        pltpu.SemaphoreType.DMA,
        pltpu.SemaphoreType.DMA,
        pltpu.SemaphoreType.DMA,
    ],
)
def _spmm_sc(src_hbm, dst_hbm, table_hbm, out_hbm,
             src_vs, dst_vs, rows_vs, stage_v, acc_sh,
             sem0, sem1, sem2, sem3, sem4):
    sems = (sem0, sem1, sem2, sem3, sem4)
    c = lax.axis_index("c")
    s = lax.axis_index("s")
    w = c * NS + s

    def _zero(i, carry):
        stage_v[i, :] = jnp.zeros((D_HID,), jnp.float32)
        return carry

    lax.fori_loop(0, ROWS_PER_TILE, _zero, 0)
    row0 = pl.multiple_of(lax.min(s * ROWS_PER_TILE, LAST_ROW0), 8)
    pltpu.sync_copy(stage_v, acc_sh.at[pl.ds(row0, ROWS_PER_TILE)])
    plsc.subcore_barrier()

    base = w * E_PER_W

    def _group(g, carry):
        idx_cps = []
        for b in range(K):
            off = base + (g * K + b) * CHUNK
            cp_s = pltpu.async_copy(src_hbm.at[pl.ds(off, CHUNK)],
                                    src_vs.at[b], sems[b])
            cp_d = pltpu.async_copy(dst_hbm.at[pl.ds(off, CHUNK)],
                                    dst_vs.at[b], sems[b])
            idx_cps.append((cp_s, cp_d))
        gathers = []
        for b in range(K):
            idx_cps[b][0].wait()
            idx_cps[b][1].wait()
            gathers.append(pltpu.async_copy(table_hbm.at[src_vs.at[b]],
                                            rows_vs.at[b], sems[b]))
        scats = []
        for b in range(K):
            gathers[b].wait()
            scats.append(pltpu.async_copy(rows_vs.at[b],
                                          acc_sh.at[dst_vs.at[b]],
                                          sems[b], add=True))
        for b in range(K):
            scats[b].wait()
        return carry

    lax.fori_loop(0, NGROUPS, _group, 0)

    plsc.subcore_barrier()
    pltpu.sync_copy(acc_sh.at[pl.ds(row0, ROWS_PER_TILE)],
                    out_hbm.at[c, pl.ds(row0, ROWS_PER_TILE)])


def _mm1_body(x_ref, w_ref, o_ref):
    o_ref[...] = jnp.dot(x_ref[...], w_ref[...],
                         preferred_element_type=jnp.float32)


def _tc_mm1(x, w):
    return pl.pallas_call(
        _mm1_body,
        out_shape=jax.ShapeDtypeStruct((N, D_HID), jnp.float32),
    )(x, w)


def _mid_body(p_ref, b1_ref, w2_ref, o_ref):
    h = jnp.maximum(p_ref[0] + p_ref[1] + b1_ref[...][None, :], 0.0)
    o_ref[...] = jnp.dot(h, w2_ref[...], preferred_element_type=jnp.float32)


def _tc_mid(parts, b1, w2p):
    return pl.pallas_call(
        _mid_body,
        out_shape=jax.ShapeDtypeStruct((N, D_HID), jnp.float32),
    )(parts, b1, w2p)


def _fin_body(p_ref, b2_ref, o_ref):
    o_ref[...] = p_ref[0] + p_ref[1] + b2_ref[...][None, :]


def _tc_fin(parts, b2p):
    return pl.pallas_call(
        _fin_body,
        out_shape=jax.ShapeDtypeStruct((N, D_HID), jnp.float32),
    )(parts, b2p)


def kernel(adjacency, feature, W1, b1, W2, b2):
    adj = adjacency.astype(jnp.int32)
    src = adj[0]
    dst = adj[1]

    support1 = _tc_mm1(feature, W1)
    parts1 = _spmm_sc(src, dst, support1)

    w2p = jnp.pad(W2, ((0, 0), (0, D_HID - D_OUT)))
    support2 = _tc_mid(parts1, b1, w2p)
    parts2 = _spmm_sc(src, dst, support2)

    b2p = jnp.pad(b2, (0, D_HID - D_OUT))
    out16 = _tc_fin(parts2, b2p)
    return out16[:, :D_OUT]


# speedup vs baseline: 21.6061x; 1.1079x over previous
import functools

import jax
import jax.numpy as jnp
from jax import lax
from jax.experimental import pallas as pl
from jax.experimental.pallas import tpu as pltpu
from jax.experimental.pallas import tpu_sc as plsc

N = 10000
E = 320000
D_IN = 128
D_HID = 16
D_OUT = 7

NC = 2
NS = 16
NW = NC * NS
E_PER_W = E // NW
CHUNK = 80
K = 5
NGROUPS = E_PER_W // (CHUNK * K)
ROWS_PER_TILE = 640
LAST_ROW0 = N - ROWS_PER_TILE


_mesh = plsc.VectorSubcoreMesh(core_axis_name="c", subcore_axis_name="s")


@functools.partial(
    pl.kernel,
    mesh=_mesh,
    out_type=jax.ShapeDtypeStruct((NC, N, D_HID), jnp.float32),
    compiler_params=pltpu.CompilerParams(use_tc_tiling_on_sc=False),
    scratch_types=[
        pltpu.VMEM((K, CHUNK), jnp.int32),
        pltpu.VMEM((K, CHUNK), jnp.int32),
        pltpu.VMEM((K, CHUNK, D_HID), jnp.float32),
        pltpu.VMEM((ROWS_PER_TILE, D_HID), jnp.float32),
        pltpu.VMEM_SHARED((N, D_HID), jnp.float32),
        pltpu.SemaphoreType.DMA,
        pltpu.SemaphoreType.DMA,
        pltpu.SemaphoreType.DMA,
        pltpu.SemaphoreType.DMA,
        pltpu.SemaphoreType.DMA,
    ],
)
def _spmm_sc(adj_hbm, table_hbm, out_hbm,
             src_vs, dst_vs, rows_vs, stage_v, acc_sh,
             sem0, sem1, sem2, sem3, sem4):
    sems = (sem0, sem1, sem2, sem3, sem4)
    c = lax.axis_index("c")
    s = lax.axis_index("s")
    w = c * NS + s

    def _zero(i, carry):
        stage_v[i, :] = jnp.zeros((D_HID,), jnp.float32)
        return carry

    lax.fori_loop(0, ROWS_PER_TILE, _zero, 0)
    row0 = pl.multiple_of(lax.min(s * ROWS_PER_TILE, LAST_ROW0), 8)
    pltpu.sync_copy(stage_v, acc_sh.at[pl.ds(row0, ROWS_PER_TILE)])
    plsc.subcore_barrier()

    base = w * E_PER_W

    def _issue_group(g, drain_prev):
        idx_cps = []
        for b in range(K):
            if drain_prev:
                pltpu.make_async_copy(rows_vs.at[b],
                                      acc_sh.at[dst_vs.at[b]],
                                      sems[b]).wait()
            off = base + (g * K + b) * CHUNK
            cp_s = pltpu.async_copy(adj_hbm.at[0, pl.ds(off, CHUNK)],
                                    src_vs.at[b], sems[b])
            cp_d = pltpu.async_copy(adj_hbm.at[1, pl.ds(off, CHUNK)],
                                    dst_vs.at[b], sems[b])
            idx_cps.append((cp_s, cp_d))
        gathers = []
        for b in range(K):
            idx_cps[b][0].wait()
            idx_cps[b][1].wait()
            gathers.append(pltpu.async_copy(table_hbm.at[src_vs.at[b]],
                                            rows_vs.at[b], sems[b]))
        for b in range(K):
            gathers[b].wait()
            pltpu.async_copy(rows_vs.at[b], acc_sh.at[dst_vs.at[b]],
                             sems[b], add=True)

    _issue_group(0, drain_prev=False)

    def _group(g, carry):
        _issue_group(g, drain_prev=True)
        return carry

    lax.fori_loop(1, NGROUPS, _group, 0)

    for b in range(K):
        pltpu.make_async_copy(rows_vs.at[b], acc_sh.at[dst_vs.at[b]],
                              sems[b]).wait()

    plsc.subcore_barrier()
    pltpu.sync_copy(acc_sh.at[pl.ds(row0, ROWS_PER_TILE)],
                    out_hbm.at[c, pl.ds(row0, ROWS_PER_TILE)])


def _mm1_body(x_ref, w_ref, o_ref):
    o_ref[...] = jnp.dot(x_ref[...], w_ref[...],
                         preferred_element_type=jnp.float32)


def _tc_mm1(x, w):
    return pl.pallas_call(
        _mm1_body,
        out_shape=jax.ShapeDtypeStruct((N, D_HID), jnp.float32),
    )(x, w)


def _mid_body(p_ref, b1_ref, w2_ref, o_ref):
    h = jnp.maximum(p_ref[0] + p_ref[1] + b1_ref[...][None, :], 0.0)
    o_ref[...] = jnp.dot(h, w2_ref[...], preferred_element_type=jnp.float32)


def _tc_mid(parts, b1, w2p):
    return pl.pallas_call(
        _mid_body,
        out_shape=jax.ShapeDtypeStruct((N, D_HID), jnp.float32),
    )(parts, b1, w2p)


def _fin_body(p_ref, b2_ref, o_ref):
    o_ref[...] = p_ref[0] + p_ref[1] + b2_ref[...][None, :]


def _tc_fin(parts, b2p):
    return pl.pallas_call(
        _fin_body,
        out_shape=jax.ShapeDtypeStruct((N, D_HID), jnp.float32),
    )(parts, b2p)


def kernel(adjacency, feature, W1, b1, W2, b2):
    adj = adjacency.astype(jnp.int32)

    support1 = _tc_mm1(feature, W1)
    parts1 = _spmm_sc(adj, support1)

    w2p = jnp.pad(W2, ((0, 0), (0, D_HID - D_OUT)))
    support2 = _tc_mid(parts1, b1, w2p)
    parts2 = _spmm_sc(adj, support2)

    b2p = jnp.pad(b2, (0, D_HID - D_OUT))
    out16 = _tc_fin(parts2, b2p)
    return out16[:, :D_OUT]


# speedup vs baseline: 24.5565x; 1.1366x over previous
import functools

import jax
import jax.numpy as jnp
from jax import lax
from jax.experimental import pallas as pl
from jax.experimental.pallas import tpu as pltpu
from jax.experimental.pallas import tpu_sc as plsc

N = 10000
E = 320000
D_IN = 128
D_HID = 16
D_OUT = 7

N_PAD = 10240
PACK = 128 // D_HID
NROWS128 = N_PAD // PACK
NC = 2
NS = 16
NW = NC * NS
E_PER_W = E // NW
CHUNK = 80
K = 5
NGROUPS = E_PER_W // (CHUNK * K)
ROWS_PER_TILE = N_PAD // NS


_mesh = plsc.VectorSubcoreMesh(core_axis_name="c", subcore_axis_name="s")


@functools.partial(
    pl.kernel,
    mesh=_mesh,
    out_type=jax.ShapeDtypeStruct((NC, NROWS128, 128), jnp.float32),
    compiler_params=pltpu.CompilerParams(use_tc_tiling_on_sc=False),
    scratch_types=[
        pltpu.VMEM((K, CHUNK), jnp.int32),
        pltpu.VMEM((K, CHUNK), jnp.int32),
        pltpu.VMEM((K, CHUNK, D_HID), jnp.float32),
        pltpu.VMEM((ROWS_PER_TILE, D_HID), jnp.float32),
        pltpu.VMEM((ROWS_PER_TILE // PACK, 128), jnp.float32),
        pltpu.VMEM_SHARED((N_PAD, D_HID), jnp.float32),
        pltpu.SemaphoreType.DMA,
        pltpu.SemaphoreType.DMA,
        pltpu.SemaphoreType.DMA,
        pltpu.SemaphoreType.DMA,
        pltpu.SemaphoreType.DMA,
    ],
)
def _spmm_sc(src_hbm, dst_hbm, table_hbm, out_hbm,
             src_vs, dst_vs, rows_vs, stage_v, stage_p, acc_sh,
             sem0, sem1, sem2, sem3, sem4):
    sems = (sem0, sem1, sem2, sem3, sem4)
    c = lax.axis_index("c")
    s = lax.axis_index("s")
    w = c * NS + s

    def _zero(i, carry):
        stage_v[i, :] = jnp.zeros((D_HID,), jnp.float32)
        return carry

    lax.fori_loop(0, ROWS_PER_TILE, _zero, 0)
    row0 = s * ROWS_PER_TILE
    pltpu.sync_copy(stage_v, acc_sh.at[pl.ds(row0, ROWS_PER_TILE)])
    plsc.subcore_barrier()

    base = w * E_PER_W

    def _issue_group(g, drain_prev):
        idx_cps = []
        for b in range(K):
            if drain_prev:
                pltpu.make_async_copy(rows_vs.at[b],
                                      acc_sh.at[dst_vs.at[b]],
                                      sems[b]).wait()
            off = base + (g * K + b) * CHUNK
            cp_s = pltpu.async_copy(src_hbm.at[pl.ds(off, CHUNK)],
                                    src_vs.at[b], sems[b])
            cp_d = pltpu.async_copy(dst_hbm.at[pl.ds(off, CHUNK)],
                                    dst_vs.at[b], sems[b])
            idx_cps.append((cp_s, cp_d))
        gathers = []
        for b in range(K):
            idx_cps[b][0].wait()
            idx_cps[b][1].wait()
            gathers.append(pltpu.async_copy(table_hbm.at[src_vs.at[b]],
                                            rows_vs.at[b], sems[b]))
        for b in range(K):
            gathers[b].wait()
            pltpu.async_copy(rows_vs.at[b], acc_sh.at[dst_vs.at[b]],
                             sems[b], add=True)

    _issue_group(0, drain_prev=False)

    def _group(g, carry):
        _issue_group(g, drain_prev=True)
        return carry

    lax.fori_loop(1, NGROUPS, _group, 0)

    for b in range(K):
        pltpu.make_async_copy(rows_vs.at[b], acc_sh.at[dst_vs.at[b]],
                              sems[b]).wait()

    plsc.subcore_barrier()

    pltpu.sync_copy(acc_sh.at[pl.ds(row0, ROWS_PER_TILE)], stage_v)

    def _repack(j, carry):
        for i in range(PACK):
            stage_p[j, pl.ds(i * D_HID, D_HID)] = stage_v[j * PACK + i, :]
        return carry

    lax.fori_loop(0, ROWS_PER_TILE // PACK, _repack, 0)
    pltpu.sync_copy(stage_p,
                    out_hbm.at[c, pl.ds(s * (ROWS_PER_TILE // PACK),
                                        ROWS_PER_TILE // PACK)])


def _mm1_body(x_ref, w_ref, o_ref):
    x3 = x_ref[...]
    w = w_ref[...]
    for i in range(PACK):
        o_ref[0:N // PACK, pl.ds(i * D_HID, D_HID)] = jnp.dot(
            x3[:, i, :], w, preferred_element_type=jnp.float32)
    o_ref[N // PACK:NROWS128, :] = jnp.zeros(
        (NROWS128 - N // PACK, 128), jnp.float32)


def _tc_mm1(x3, w):
    return pl.pallas_call(
        _mm1_body,
        out_shape=jax.ShapeDtypeStruct((NROWS128, 128), jnp.float32),
    )(x3, w)


def _mid_body(p_ref, b1r_ref, bd_ref, o_ref):
    h = jnp.maximum(p_ref[0] + p_ref[1] + b1r_ref[...][None, :], 0.0)
    o_ref[...] = jnp.dot(h, bd_ref[...], preferred_element_type=jnp.float32)


def _tc_mid(parts, b1r, bd):
    return pl.pallas_call(
        _mid_body,
        out_shape=jax.ShapeDtypeStruct((NROWS128, 128), jnp.float32),
    )(parts, b1r, bd)


def _fin_body(p_ref, b2r_ref, o_ref):
    o_ref[...] = p_ref[0, 0:N // PACK, :] + p_ref[1, 0:N // PACK, :] \
        + b2r_ref[...][None, :]


def _tc_fin(parts, b2r):
    return pl.pallas_call(
        _fin_body,
        out_shape=jax.ShapeDtypeStruct((N // PACK, 128), jnp.float32),
    )(parts, b2r)


def kernel(adjacency, feature, W1, b1, W2, b2):
    adj = adjacency.astype(jnp.int32)
    src = adj[0]
    dst = adj[1]

    x3 = feature.reshape(N // PACK, PACK, D_IN)
    packed1 = _tc_mm1(x3, W1)
    table1 = packed1.reshape(N_PAD, D_HID)
    parts1 = _spmm_sc(src, dst, table1)

    w2p = jnp.pad(W2, ((0, 0), (0, D_HID - D_OUT)))
    b1r = jnp.tile(b1, PACK)
    bd = jnp.kron(jnp.eye(PACK, dtype=jnp.float32), w2p)
    packed2 = _tc_mid(parts1, b1r, bd)
    table2 = packed2.reshape(N_PAD, D_HID)
    parts2 = _spmm_sc(src, dst, table2)

    b2r = jnp.tile(jnp.pad(b2, (0, D_HID - D_OUT)), PACK)
    out128 = _tc_fin(parts2, b2r)
    return out128.reshape(N, D_HID)[:, :D_OUT]


# speedup vs baseline: 26.7286x; 1.0885x over previous
import functools

import jax
import jax.numpy as jnp
from jax import lax
from jax.experimental import pallas as pl
from jax.experimental.pallas import tpu as pltpu
from jax.experimental.pallas import tpu_sc as plsc

N = 10000
E = 320000
D_IN = 128
D_HID = 16
D_OUT = 7

N_PAD = 10240
PACK = 128 // D_HID
NROWS128 = N_PAD // PACK
NC = 2
NS = 16
NW = NC * NS
E_PER_W = E // NW
CHUNK = 80
K = 5
NGROUPS = E_PER_W // (CHUNK * K)
ROWS_PER_TILE = N_PAD // NS


_mesh = plsc.VectorSubcoreMesh(core_axis_name="c", subcore_axis_name="s")


@functools.partial(
    pl.kernel,
    mesh=_mesh,
    out_type=jax.ShapeDtypeStruct((NC, NROWS128, 128), jnp.float32),
    compiler_params=pltpu.CompilerParams(use_tc_tiling_on_sc=False),
    scratch_types=[
        pltpu.VMEM((K, 2, CHUNK), jnp.int32),
        pltpu.VMEM((K, CHUNK, D_HID), jnp.float32),
        pltpu.VMEM((ROWS_PER_TILE, D_HID), jnp.float32),
        pltpu.VMEM((ROWS_PER_TILE // PACK, 128), jnp.float32),
        pltpu.VMEM_SHARED((N_PAD, D_HID), jnp.float32),
        pltpu.SemaphoreType.DMA,
        pltpu.SemaphoreType.DMA,
        pltpu.SemaphoreType.DMA,
        pltpu.SemaphoreType.DMA,
        pltpu.SemaphoreType.DMA,
    ],
)
def _spmm_sc(adj_hbm, table_hbm, out_hbm,
             idx_vs, rows_vs, stage_v, stage_p, acc_sh,
             sem0, sem1, sem2, sem3, sem4):
    sems = (sem0, sem1, sem2, sem3, sem4)
    c = lax.axis_index("c")
    s = lax.axis_index("s")
    w = c * NS + s

    def _zero(i, carry):
        stage_v[i, :] = jnp.zeros((D_HID,), jnp.float32)
        return carry

    lax.fori_loop(0, ROWS_PER_TILE, _zero, 0)
    row0 = s * ROWS_PER_TILE
    pltpu.sync_copy(stage_v, acc_sh.at[pl.ds(row0, ROWS_PER_TILE)])
    plsc.subcore_barrier()

    base = w * E_PER_W

    def _issue_group(g, drain_prev):
        idx_cps = []
        for b in range(K):
            if drain_prev:
                pltpu.make_async_copy(rows_vs.at[b],
                                      acc_sh.at[idx_vs.at[b, 1]],
                                      sems[b]).wait()
            off = base + (g * K + b) * CHUNK
            idx_cps.append(
                pltpu.async_copy(adj_hbm.at[:, pl.ds(off, CHUNK)],
                                 idx_vs.at[b], sems[b]))
        gathers = []
        for b in range(K):
            idx_cps[b].wait()
            gathers.append(pltpu.async_copy(table_hbm.at[idx_vs.at[b, 0]],
                                            rows_vs.at[b], sems[b]))
        for b in range(K):
            gathers[b].wait()
            pltpu.async_copy(rows_vs.at[b], acc_sh.at[idx_vs.at[b, 1]],
                             sems[b], add=True)

    _issue_group(0, drain_prev=False)

    def _group(g, carry):
        _issue_group(g, drain_prev=True)
        return carry

    lax.fori_loop(1, NGROUPS, _group, 0)

    for b in range(K):
        pltpu.make_async_copy(rows_vs.at[b], acc_sh.at[idx_vs.at[b, 1]],
                              sems[b]).wait()

    plsc.subcore_barrier()

    pltpu.sync_copy(acc_sh.at[pl.ds(row0, ROWS_PER_TILE)], stage_v)

    def _repack(j, carry):
        for i in range(PACK):
            stage_p[j, pl.ds(i * D_HID, D_HID)] = stage_v[j * PACK + i, :]
        return carry

    lax.fori_loop(0, ROWS_PER_TILE // PACK, _repack, 0)
    pltpu.sync_copy(stage_p,
                    out_hbm.at[c, pl.ds(s * (ROWS_PER_TILE // PACK),
                                        ROWS_PER_TILE // PACK)])


def _mm1_body(x_ref, w_ref, o_ref):
    x3 = x_ref[...]
    w = w_ref[...]
    for i in range(PACK):
        o_ref[0:N // PACK, pl.ds(i * D_HID, D_HID)] = jnp.dot(
            x3[:, i, :], w, preferred_element_type=jnp.float32)
    o_ref[N // PACK:NROWS128, :] = jnp.zeros(
        (NROWS128 - N // PACK, 128), jnp.float32)


def _tc_mm1(x3, w):
    return pl.pallas_call(
        _mm1_body,
        out_shape=jax.ShapeDtypeStruct((NROWS128, 128), jnp.float32),
    )(x3, w)


def _mid_body(p_ref, b1r_ref, bd_ref, o_ref):
    h = jnp.maximum(p_ref[0] + p_ref[1] + b1r_ref[...][None, :], 0.0)
    o_ref[...] = jnp.dot(h, bd_ref[...], preferred_element_type=jnp.float32)


def _tc_mid(parts, b1r, bd):
    return pl.pallas_call(
        _mid_body,
        out_shape=jax.ShapeDtypeStruct((NROWS128, 128), jnp.float32),
    )(parts, b1r, bd)


def _fin_body(p_ref, b2r_ref, o_ref):
    o_ref[...] = p_ref[0, 0:N // PACK, :] + p_ref[1, 0:N // PACK, :] \
        + b2r_ref[...][None, :]


def _tc_fin(parts, b2r):
    return pl.pallas_call(
        _fin_body,
        out_shape=jax.ShapeDtypeStruct((N // PACK, 128), jnp.float32),
    )(parts, b2r)


def kernel(adjacency, feature, W1, b1, W2, b2):
    adj = adjacency.astype(jnp.int32)

    x3 = feature.reshape(N // PACK, PACK, D_IN)
    packed1 = _tc_mm1(x3, W1)
    table1 = packed1.reshape(N_PAD, D_HID)
    parts1 = _spmm_sc(adj, table1)

    w2p = jnp.pad(W2, ((0, 0), (0, D_HID - D_OUT)))
    b1r = jnp.tile(b1, PACK)
    bd = jnp.kron(jnp.eye(PACK, dtype=jnp.float32), w2p)
    packed2 = _tc_mid(parts1, b1r, bd)
    table2 = packed2.reshape(N_PAD, D_HID)
    parts2 = _spmm_sc(adj, table2)

    b2r = jnp.tile(jnp.pad(b2, (0, D_HID - D_OUT)), PACK)
    out128 = _tc_fin(parts2, b2r)
    return out128.reshape(N, D_HID)[:, :D_OUT]


# speedup vs baseline: 32.9583x; 1.2331x over previous
import functools

import jax
import jax.numpy as jnp
from jax import lax
from jax.experimental import pallas as pl
from jax.experimental.pallas import tpu as pltpu
from jax.experimental.pallas import tpu_sc as plsc

N = 10000
E = 320000
D_IN = 128
D_HID = 16
D_OUT = 7

N_PAD = 10240
PACK = 128 // D_HID
NROWS128 = N_PAD // PACK
NC = 2
NS = 16
NW = NC * NS
E_PER_W = E // NW
CHUNK = 128
K = 6
NGROUPS = 13
TAIL = E_PER_W - NGROUPS * K * CHUNK
ROWS_PER_TILE = N_PAD // NS


_mesh = plsc.VectorSubcoreMesh(core_axis_name="c", subcore_axis_name="s")


@functools.partial(
    pl.kernel,
    mesh=_mesh,
    out_type=jax.ShapeDtypeStruct((NC, NROWS128, 128), jnp.float32),
    compiler_params=pltpu.CompilerParams(use_tc_tiling_on_sc=False),
    scratch_types=[
        pltpu.VMEM((K, 2, CHUNK), jnp.int32),
        pltpu.VMEM((K, CHUNK, D_HID), jnp.float32),
        pltpu.VMEM((2, TAIL), jnp.int32),
        pltpu.VMEM((TAIL, D_HID), jnp.float32),
        pltpu.VMEM((ROWS_PER_TILE, D_HID), jnp.float32),
        pltpu.VMEM((ROWS_PER_TILE // PACK, 128), jnp.float32),
        pltpu.VMEM_SHARED((N_PAD, D_HID), jnp.float32),
        pltpu.SemaphoreType.DMA,
        pltpu.SemaphoreType.DMA,
        pltpu.SemaphoreType.DMA,
        pltpu.SemaphoreType.DMA,
        pltpu.SemaphoreType.DMA,
        pltpu.SemaphoreType.DMA,
        pltpu.SemaphoreType.DMA,
    ],
)
def _spmm_sc(adj_hbm, table_hbm, out_hbm,
             idx_vs, rows_vs, idx_t, rows_t, stage_v, stage_p, acc_sh,
             sem0, sem1, sem2, sem3, sem4, sem5, sem6):
    sems = (sem0, sem1, sem2, sem3, sem4, sem5)
    c = lax.axis_index("c")
    s = lax.axis_index("s")
    w = c * NS + s

    def _zero(i, carry):
        stage_v[i, :] = jnp.zeros((D_HID,), jnp.float32)
        return carry

    lax.fori_loop(0, ROWS_PER_TILE, _zero, 0)
    row0 = s * ROWS_PER_TILE
    pltpu.sync_copy(stage_v, acc_sh.at[pl.ds(row0, ROWS_PER_TILE)])
    plsc.subcore_barrier()

    base = w * E_PER_W

    def _issue_group(g, drain_prev):
        idx_cps = []
        for b in range(K):
            if drain_prev:
                pltpu.make_async_copy(rows_vs.at[b],
                                      acc_sh.at[idx_vs.at[b, 1]],
                                      sems[b]).wait()
            off = base + (g * K + b) * CHUNK
            idx_cps.append(
                pltpu.async_copy(adj_hbm.at[:, pl.ds(off, CHUNK)],
                                 idx_vs.at[b], sems[b]))
        gathers = []
        for b in range(K):
            idx_cps[b].wait()
            gathers.append(pltpu.async_copy(table_hbm.at[idx_vs.at[b, 0]],
                                            rows_vs.at[b], sems[b]))
        for b in range(K):
            gathers[b].wait()
            pltpu.async_copy(rows_vs.at[b], acc_sh.at[idx_vs.at[b, 1]],
                             sems[b], add=True)

    _issue_group(0, drain_prev=False)

    def _group(g, carry):
        _issue_group(g, drain_prev=True)
        return carry

    lax.fori_loop(1, NGROUPS, _group, 0)

    toff = base + NGROUPS * K * CHUNK
    pltpu.async_copy(adj_hbm.at[:, pl.ds(toff, TAIL)], idx_t, sem6).wait()
    pltpu.async_copy(table_hbm.at[idx_t.at[0]], rows_t, sem6).wait()
    pltpu.async_copy(rows_t, acc_sh.at[idx_t.at[1]], sem6, add=True)

    for b in range(K):
        pltpu.make_async_copy(rows_vs.at[b], acc_sh.at[idx_vs.at[b, 1]],
                              sems[b]).wait()
    pltpu.make_async_copy(rows_t, acc_sh.at[idx_t.at[1]], sem6).wait()

    plsc.subcore_barrier()

    pltpu.sync_copy(acc_sh.at[pl.ds(row0, ROWS_PER_TILE)], stage_v)

    def _repack(j, carry):
        for i in range(PACK):
            stage_p[j, pl.ds(i * D_HID, D_HID)] = stage_v[j * PACK + i, :]
        return carry

    lax.fori_loop(0, ROWS_PER_TILE // PACK, _repack, 0)
    pltpu.sync_copy(stage_p,
                    out_hbm.at[c, pl.ds(s * (ROWS_PER_TILE // PACK),
                                        ROWS_PER_TILE // PACK)])


def _mm1_body(x_ref, w_ref, o_ref):
    x3 = x_ref[...]
    w = w_ref[...]
    for i in range(PACK):
        o_ref[0:N // PACK, pl.ds(i * D_HID, D_HID)] = jnp.dot(
            x3[:, i, :], w, preferred_element_type=jnp.float32)
    o_ref[N // PACK:NROWS128, :] = jnp.zeros(
        (NROWS128 - N // PACK, 128), jnp.float32)


def _tc_mm1(x3, w):
    return pl.pallas_call(
        _mm1_body,
        out_shape=jax.ShapeDtypeStruct((NROWS128, 128), jnp.float32),
    )(x3, w)


def _mid_body(p_ref, b1r_ref, bd_ref, o_ref):
    h = jnp.maximum(p_ref[0] + p_ref[1] + b1r_ref[...][None, :], 0.0)
    o_ref[...] = jnp.dot(h, bd_ref[...], preferred_element_type=jnp.float32)


def _tc_mid(parts, b1r, bd):
    return pl.pallas_call(
        _mid_body,
        out_shape=jax.ShapeDtypeStruct((NROWS128, 128), jnp.float32),
    )(parts, b1r, bd)


def _fin_body(p_ref, b2r_ref, o_ref):
    o_ref[...] = p_ref[0, 0:N // PACK, :] + p_ref[1, 0:N // PACK, :] \
        + b2r_ref[...][None, :]


def _tc_fin(parts, b2r):
    return pl.pallas_call(
        _fin_body,
        out_shape=jax.ShapeDtypeStruct((N // PACK, 128), jnp.float32),
    )(parts, b2r)


def kernel(adjacency, feature, W1, b1, W2, b2):
    adj = adjacency

    x3 = feature.reshape(N // PACK, PACK, D_IN)
    packed1 = _tc_mm1(x3, W1)
    table1 = packed1.reshape(N_PAD, D_HID)
    parts1 = _spmm_sc(adj, table1)

    w2p = jnp.pad(W2, ((0, 0), (0, D_HID - D_OUT)))
    b1r = jnp.tile(b1, PACK)
    bd = jnp.kron(jnp.eye(PACK, dtype=jnp.float32), w2p)
    packed2 = _tc_mid(parts1, b1r, bd)
    table2 = packed2.reshape(N_PAD, D_HID)
    parts2 = _spmm_sc(adj, table2)

    b2r = jnp.tile(jnp.pad(b2, (0, D_HID - D_OUT)), PACK)
    out128 = _tc_fin(parts2, b2r)
    return out128.reshape(N, D_HID)[:, :D_OUT]


# speedup vs baseline: 36.2634x; 1.1003x over previous
import functools

import jax
import jax.numpy as jnp
from jax import lax
from jax.experimental import pallas as pl
from jax.experimental.pallas import tpu as pltpu
from jax.experimental.pallas import tpu_sc as plsc

N = 10000
E = 320000
D_IN = 128
D_HID = 16
D_OUT = 7

N_PAD = 10240
PACK = 128 // D_HID
NROWS128 = N_PAD // PACK
NC = 2
NS = 16
NW = NC * NS
E_PER_W = E // NW
CHUNK = 128
K = 6
NGROUPS = 13
TAIL = E_PER_W - NGROUPS * K * CHUNK
ROWS_PER_TILE = N_PAD // NS


_mesh = plsc.VectorSubcoreMesh(core_axis_name="c", subcore_axis_name="s")


@functools.partial(
    pl.kernel,
    mesh=_mesh,
    out_type=jax.ShapeDtypeStruct((NC, NROWS128, 128), jnp.float32),
    compiler_params=pltpu.CompilerParams(use_tc_tiling_on_sc=False),
    scratch_types=[
        pltpu.VMEM((E_PER_W,), jnp.int32),
        pltpu.VMEM((NGROUPS * K, CHUNK), jnp.int32),
        pltpu.VMEM((K, CHUNK, D_HID), jnp.float32),
        pltpu.VMEM((2, TAIL), jnp.int32),
        pltpu.VMEM((TAIL, D_HID), jnp.float32),
        pltpu.VMEM((ROWS_PER_TILE, D_HID), jnp.float32),
        pltpu.VMEM((ROWS_PER_TILE // PACK, 128), jnp.float32),
        pltpu.VMEM_SHARED((N_PAD, D_HID), jnp.float32),
        pltpu.SemaphoreType.DMA,
        pltpu.SemaphoreType.DMA,
        pltpu.SemaphoreType.DMA,
        pltpu.SemaphoreType.DMA,
        pltpu.SemaphoreType.DMA,
        pltpu.SemaphoreType.DMA,
        pltpu.SemaphoreType.DMA,
    ],
)
def _spmm_sc(adj_hbm, table_hbm, out_hbm,
             src_all, dst_all, rows_vs, idx_t, rows_t, stage_v, stage_p,
             acc_sh, sem0, sem1, sem2, sem3, sem4, sem5, sem6):
    sems = (sem0, sem1, sem2, sem3, sem4, sem5)
    c = lax.axis_index("c")
    s = lax.axis_index("s")
    w = c * NS + s
    base = w * E_PER_W

    pre_src = pltpu.async_copy(adj_hbm.at[0, pl.ds(base, E_PER_W)],
                               src_all, sem6)
    pre_dst = []
    for ch in range(NGROUPS * K):
        pre_dst.append(
            pltpu.async_copy(adj_hbm.at[1, pl.ds(base + ch * CHUNK, CHUNK)],
                             dst_all.at[ch], sem6))

    def _zero(i, carry):
        stage_v[i, :] = jnp.zeros((D_HID,), jnp.float32)
        return carry

    lax.fori_loop(0, ROWS_PER_TILE, _zero, 0)
    row0 = s * ROWS_PER_TILE
    pltpu.sync_copy(stage_v, acc_sh.at[pl.ds(row0, ROWS_PER_TILE)])

    pre_src.wait()
    for cp in pre_dst:
        cp.wait()
    plsc.subcore_barrier()

    def _issue_group(g, drain_prev):
        gathers = []
        for b in range(K):
            ch = g * K + b
            if drain_prev:
                pltpu.make_async_copy(rows_vs.at[b],
                                      acc_sh.at[dst_all.at[ch - K]],
                                      sems[b]).wait()
            gathers.append(pltpu.async_copy(
                table_hbm.at[src_all.at[pl.ds(ch * CHUNK, CHUNK)]],
                rows_vs.at[b], sems[b]))
        for b in range(K):
            ch = g * K + b
            gathers[b].wait()
            pltpu.async_copy(rows_vs.at[b], acc_sh.at[dst_all.at[ch]],
                             sems[b], add=True)

    _issue_group(0, drain_prev=False)

    def _group(g, carry):
        _issue_group(g, drain_prev=True)
        return carry

    lax.fori_loop(1, NGROUPS, _group, 0)

    toff = base + NGROUPS * K * CHUNK
    pltpu.async_copy(adj_hbm.at[:, pl.ds(toff, TAIL)], idx_t, sem6).wait()
    pltpu.async_copy(table_hbm.at[idx_t.at[0]], rows_t, sem6).wait()
    pltpu.async_copy(rows_t, acc_sh.at[idx_t.at[1]], sem6, add=True)

    for b in range(K):
        ch = (NGROUPS - 1) * K + b
        pltpu.make_async_copy(rows_vs.at[b], acc_sh.at[dst_all.at[ch]],
                              sems[b]).wait()
    pltpu.make_async_copy(rows_t, acc_sh.at[idx_t.at[1]], sem6).wait()

    plsc.subcore_barrier()

    pltpu.sync_copy(acc_sh.at[pl.ds(row0, ROWS_PER_TILE)], stage_v)

    def _repack(j, carry):
        for i in range(PACK):
            stage_p[j, pl.ds(i * D_HID, D_HID)] = stage_v[j * PACK + i, :]
        return carry

    lax.fori_loop(0, ROWS_PER_TILE // PACK, _repack, 0)
    pltpu.sync_copy(stage_p,
                    out_hbm.at[c, pl.ds(s * (ROWS_PER_TILE // PACK),
                                        ROWS_PER_TILE // PACK)])


def _mm1_body(x_ref, w_ref, o_ref):
    x3 = x_ref[...]
    w = w_ref[...]
    for i in range(PACK):
        o_ref[0:N // PACK, pl.ds(i * D_HID, D_HID)] = jnp.dot(
            x3[:, i, :], w, preferred_element_type=jnp.float32)
    o_ref[N // PACK:NROWS128, :] = jnp.zeros(
        (NROWS128 - N // PACK, 128), jnp.float32)


def _tc_mm1(x3, w):
    return pl.pallas_call(
        _mm1_body,
        out_shape=jax.ShapeDtypeStruct((NROWS128, 128), jnp.float32),
    )(x3, w)


def _mid_body(p_ref, b1r_ref, bd_ref, o_ref):
    h = jnp.maximum(p_ref[0] + p_ref[1] + b1r_ref[...][None, :], 0.0)
    o_ref[...] = jnp.dot(h, bd_ref[...], preferred_element_type=jnp.float32)


def _tc_mid(parts, b1r, bd):
    return pl.pallas_call(
        _mid_body,
        out_shape=jax.ShapeDtypeStruct((NROWS128, 128), jnp.float32),
    )(parts, b1r, bd)


def _fin_body(p_ref, b2r_ref, o_ref):
    o_ref[...] = p_ref[0, 0:N // PACK, :] + p_ref[1, 0:N // PACK, :] \
        + b2r_ref[...][None, :]


def _tc_fin(parts, b2r):
    return pl.pallas_call(
        _fin_body,
        out_shape=jax.ShapeDtypeStruct((N // PACK, 128), jnp.float32),
    )(parts, b2r)


def kernel(adjacency, feature, W1, b1, W2, b2):
    adj = adjacency

    x3 = feature.reshape(N // PACK, PACK, D_IN)
    packed1 = _tc_mm1(x3, W1)
    table1 = packed1.reshape(N_PAD, D_HID)
    parts1 = _spmm_sc(adj, table1)

    w2p = jnp.pad(W2, ((0, 0), (0, D_HID - D_OUT)))
    b1r = jnp.tile(b1, PACK)
    bd = jnp.kron(jnp.eye(PACK, dtype=jnp.float32), w2p)
    packed2 = _tc_mid(parts1, b1r, bd)
    table2 = packed2.reshape(N_PAD, D_HID)
    parts2 = _spmm_sc(adj, table2)

    b2r = jnp.tile(jnp.pad(b2, (0, D_HID - D_OUT)), PACK)
    out128 = _tc_fin(parts2, b2r)
    return out128.reshape(N, D_HID)[:, :D_OUT]
